# baseline (speedup 1.0000x reference)
import jax
import jax.numpy as jnp
from jax.experimental import pallas as pl
from jax.experimental.pallas import tpu as pltpu

_OUT = 32
_GMAX = 4
_HW = 128
_CBLK = 64
_S = 8


def _interp_matrix(c1, c2):
    roi = jnp.maximum(c2 - c1, 1.0)
    binsz = roi / float(_OUT)
    g_cnt = jnp.maximum(jnp.ceil(roi / float(_OUT)), 1.0)
    p = jax.lax.broadcasted_iota(jnp.int32, (_OUT, _GMAX), 0).astype(jnp.float32)
    g = jax.lax.broadcasted_iota(jnp.int32, (_OUT, _GMAX), 1).astype(jnp.float32)
    cc = c1 + p * binsz + (g + 0.5) * binsz / g_cnt
    valid = (g < g_cnt) & (cc > -1.0) & (cc < float(_HW))
    c = jnp.clip(cc, 0.0, float(_HW - 1))
    c0 = jnp.floor(c)
    c0i = c0.astype(jnp.int32)
    c1i = jnp.minimum(c0i + 1, _HW - 1)
    l = c - c0
    h = 1.0 - l
    w = jnp.where(valid, 1.0, 0.0) / g_cnt
    k = jax.lax.broadcasted_iota(jnp.int32, (_OUT, _GMAX, _HW), 2)
    m = (jnp.where(k == c0i[:, :, None], (h * w)[:, :, None], 0.0)
         + jnp.where(k == c1i[:, :, None], (l * w)[:, :, None], 0.0))
    return m.sum(axis=1)


def _box_mats(m):
    row_any = jnp.any(m, axis=1, keepdims=True)
    col_any = jnp.any(m, axis=0, keepdims=True)
    ri = jax.lax.broadcasted_iota(jnp.int32, (_HW, 1), 0)
    ci = jax.lax.broadcasted_iota(jnp.int32, (1, _HW), 1)
    x1 = jnp.min(jnp.where(row_any, ri, _HW))
    x2 = jnp.max(jnp.where(row_any, ri, -1))
    y1 = jnp.min(jnp.where(col_any, ci, _HW))
    y2 = jnp.max(jnp.where(col_any, ci, -1))
    x_len = x2 - x1
    y_len = y2 - y1
    half = jnp.abs(x_len - y_len) // 2
    grow_y = x_len > y_len
    y1 = jnp.where(grow_y, y1 - half, y1)
    y2 = jnp.where(grow_y, y2 + half, y2)
    x1 = jnp.where(grow_y, x1, x1 - half)
    x2 = jnp.where(grow_y, x2, x2 + half)
    a_mat = _interp_matrix(x1.astype(jnp.float32), x2.astype(jnp.float32))
    b_mat = _interp_matrix(y1.astype(jnp.float32), y2.astype(jnp.float32))
    return a_mat, b_mat


def _crop_kernel(m_ref, f_ref, o_ref):
    a_list, b_list = [], []
    for s in range(_S):
        a_mat, b_mat = _box_mats(m_ref[0, s])
        a_list.append(a_mat)
        b_list.append(b_mat)
    a_all = jnp.concatenate(a_list, axis=0)
    f = f_ref[0]
    r1 = jax.lax.dot_general(f, a_all, (((1,), (1,)), ((), ())),
                             preferred_element_type=jnp.float32)
    for s in range(_S):
        sl = jax.lax.slice(r1, (0, 0, _OUT * s),
                           (_CBLK, _HW, _OUT * (s + 1)))
        o_ref[0, s] = jax.lax.dot_general(sl, b_list[s],
                                          (((1,), (1,)), ((), ())),
                                          preferred_element_type=jnp.float32)


def kernel(encoder_features, depths, intrinsics, masks):
    del depths, intrinsics
    n, c, _, _ = encoder_features.shape
    ms = masks[:, :, ::4, ::4]
    grid = (n, c // _CBLK)
    out = pl.pallas_call(
        _crop_kernel,
        grid=grid,
        in_specs=[
            pl.BlockSpec((1, _S, _HW, _HW), lambda b, cb: (b, 0, 0, 0)),
            pl.BlockSpec((1, _CBLK, _HW, _HW), lambda b, cb: (b, cb, 0, 0)),
        ],
        out_specs=pl.BlockSpec((1, _S, _CBLK, _OUT, _OUT),
                               lambda b, cb: (b, 0, cb, 0, 0)),
        out_shape=jax.ShapeDtypeStruct((n, _S, c, _OUT, _OUT), jnp.float32),
    )(ms, encoder_features)
    return out


# speedup vs baseline: 1.5536x; 1.5536x over previous
import jax
import jax.numpy as jnp
from jax.experimental import pallas as pl
from jax.experimental.pallas import tpu as pltpu

_OUT = 32
_GMAX = 4
_HW = 128
_CBLK = 128
_S = 8


def _interp_matrix(c1, c2):
    roi = jnp.maximum(c2 - c1, 1.0)
    binsz = roi / float(_OUT)
    g_cnt = jnp.maximum(jnp.ceil(roi / float(_OUT)), 1.0)
    p = jax.lax.broadcasted_iota(jnp.int32, (_OUT, _GMAX), 0).astype(jnp.float32)
    g = jax.lax.broadcasted_iota(jnp.int32, (_OUT, _GMAX), 1).astype(jnp.float32)
    cc = c1 + p * binsz + (g + 0.5) * binsz / g_cnt
    valid = (g < g_cnt) & (cc > -1.0) & (cc < float(_HW))
    c = jnp.clip(cc, 0.0, float(_HW - 1))
    c0 = jnp.floor(c)
    c0i = c0.astype(jnp.int32)
    c1i = jnp.minimum(c0i + 1, _HW - 1)
    l = c - c0
    h = 1.0 - l
    w = jnp.where(valid, 1.0, 0.0) / g_cnt
    k = jax.lax.broadcasted_iota(jnp.int32, (_OUT, _GMAX, _HW), 2)
    m = (jnp.where(k == c0i[:, :, None], (h * w)[:, :, None], 0.0)
         + jnp.where(k == c1i[:, :, None], (l * w)[:, :, None], 0.0))
    return m.sum(axis=1)


def _box_mats(m):
    row_any = jnp.any(m, axis=1, keepdims=True)
    col_any = jnp.any(m, axis=0, keepdims=True)
    ri = jax.lax.broadcasted_iota(jnp.int32, (_HW, 1), 0)
    ci = jax.lax.broadcasted_iota(jnp.int32, (1, _HW), 1)
    x1 = jnp.min(jnp.where(row_any, ri, _HW))
    x2 = jnp.max(jnp.where(row_any, ri, -1))
    y1 = jnp.min(jnp.where(col_any, ci, _HW))
    y2 = jnp.max(jnp.where(col_any, ci, -1))
    x_len = x2 - x1
    y_len = y2 - y1
    half = jnp.abs(x_len - y_len) // 2
    grow_y = x_len > y_len
    y1 = jnp.where(grow_y, y1 - half, y1)
    y2 = jnp.where(grow_y, y2 + half, y2)
    x1 = jnp.where(grow_y, x1, x1 - half)
    x2 = jnp.where(grow_y, x2, x2 + half)
    a_mat = _interp_matrix(x1.astype(jnp.float32), x2.astype(jnp.float32))
    b_mat = _interp_matrix(y1.astype(jnp.float32), y2.astype(jnp.float32))
    return a_mat, b_mat


def _crop_kernel(m_ref, f_ref, o_ref):
    a_list, b_list = [], []
    for s in range(_S):
        a_mat, b_mat = _box_mats(m_ref[0, s])
        a_list.append(a_mat)
        b_list.append(b_mat)
    a_all = jnp.concatenate(a_list, axis=0).astype(jnp.bfloat16)
    f = f_ref[0].astype(jnp.bfloat16)
    r1 = jax.lax.dot_general(a_all, f, (((1,), (1,)), ((), ())),
                             preferred_element_type=jnp.float32)
    for s in range(_S):
        sl = jax.lax.slice(r1, (_OUT * s, 0, 0),
                           (_OUT * (s + 1), _CBLK, _HW))
        r2 = jax.lax.dot_general(sl.reshape(_OUT * _CBLK, _HW), b_list[s],
                                 (((1,), (1,)), ((), ())),
                                 preferred_element_type=jnp.float32)
        o_ref[0, s] = (r2.reshape(_OUT, _CBLK, _OUT).transpose(1, 0, 2)
                       .reshape(_CBLK, _OUT * _OUT))


def kernel(encoder_features, depths, intrinsics, masks):
    del depths, intrinsics
    n, c, _, _ = encoder_features.shape
    ms = masks[:, :, ::4, ::4]
    grid = (n, c // _CBLK)
    out = pl.pallas_call(
        _crop_kernel,
        grid=grid,
        in_specs=[
            pl.BlockSpec((1, _S, _HW, _HW), lambda b, cb: (b, 0, 0, 0)),
            pl.BlockSpec((1, _CBLK, _HW, _HW), lambda b, cb: (b, cb, 0, 0)),
        ],
        out_specs=pl.BlockSpec((1, _S, _CBLK, _OUT * _OUT),
                               lambda b, cb: (b, 0, cb, 0)),
        out_shape=jax.ShapeDtypeStruct((n, _S, c, _OUT * _OUT), jnp.float32),
    )(ms, encoder_features)
    return out.reshape(n, _S, c, _OUT, _OUT)


# speedup vs baseline: 2.0959x; 1.3490x over previous
import jax
import jax.numpy as jnp
from jax.experimental import pallas as pl
from jax.experimental.pallas import tpu as pltpu

_OUT = 32
_GMAX = 4
_HW = 128
_CBLK = 128
_S = 8


def _interp_matrix(c1, c2):
    roi = jnp.maximum(c2 - c1, 1.0)
    binsz = roi / float(_OUT)
    g_cnt = jnp.maximum(jnp.ceil(roi / float(_OUT)), 1.0)
    p = jax.lax.broadcasted_iota(jnp.int32, (_OUT, _GMAX), 0).astype(jnp.float32)
    g = jax.lax.broadcasted_iota(jnp.int32, (_OUT, _GMAX), 1).astype(jnp.float32)
    cc = c1 + p * binsz + (g + 0.5) * binsz / g_cnt
    valid = (g < g_cnt) & (cc > -1.0) & (cc < float(_HW))
    c = jnp.clip(cc, 0.0, float(_HW - 1))
    c0 = jnp.floor(c)
    c0i = c0.astype(jnp.int32)
    c1i = jnp.minimum(c0i + 1, _HW - 1)
    l = c - c0
    h = 1.0 - l
    w = jnp.where(valid, 1.0, 0.0) / g_cnt
    k = jax.lax.broadcasted_iota(jnp.int32, (_OUT, _GMAX, _HW), 2)
    m = (jnp.where(k == c0i[:, :, None], (h * w)[:, :, None], 0.0)
         + jnp.where(k == c1i[:, :, None], (l * w)[:, :, None], 0.0))
    return m.sum(axis=1)


def _box_mats(m):
    m32 = m.astype(jnp.int32)
    li = jax.lax.broadcasted_iota(jnp.int32, (1, 4 * _HW), 1)
    keep = (li % 4) == 0
    keep_f = (jax.lax.broadcasted_iota(jnp.int32, (_HW, 4 * _HW), 1) % 4) == 0
    row_any = jnp.max(jnp.where(keep_f, m32, 0), axis=1, keepdims=True) > 0
    col_any = (jnp.max(m32, axis=0, keepdims=True) > 0) & keep
    ri = jax.lax.broadcasted_iota(jnp.int32, (_HW, 1), 0)
    ci = li // 4
    x1 = jnp.min(jnp.where(row_any, ri, _HW))
    x2 = jnp.max(jnp.where(row_any, ri, -1))
    y1 = jnp.min(jnp.where(col_any, ci, _HW))
    y2 = jnp.max(jnp.where(col_any, ci, -1))
    x_len = x2 - x1
    y_len = y2 - y1
    half = jnp.abs(x_len - y_len) // 2
    grow_y = x_len > y_len
    y1 = jnp.where(grow_y, y1 - half, y1)
    y2 = jnp.where(grow_y, y2 + half, y2)
    x1 = jnp.where(grow_y, x1, x1 - half)
    x2 = jnp.where(grow_y, x2, x2 + half)
    a_mat = _interp_matrix(x1.astype(jnp.float32), x2.astype(jnp.float32))
    b_mat = _interp_matrix(y1.astype(jnp.float32), y2.astype(jnp.float32))
    return a_mat, b_mat


def _crop_kernel(m_ref, f_ref, o_ref):
    a_list, b_list = [], []
    for s in range(_S):
        a_mat, b_mat = _box_mats(m_ref[0, s, 0, :, :])
        a_list.append(a_mat)
        b_list.append(b_mat)
    a_all = jnp.concatenate(a_list, axis=0).astype(jnp.bfloat16)
    f = f_ref[0].astype(jnp.bfloat16)
    r1 = jax.lax.dot_general(a_all, f, (((1,), (1,)), ((), ())),
                             preferred_element_type=jnp.float32)
    for s in range(_S):
        sl = jax.lax.slice(r1, (_OUT * s, 0, 0),
                           (_OUT * (s + 1), _CBLK, _HW))
        r2 = jax.lax.dot_general(sl.reshape(_OUT * _CBLK, _HW), b_list[s],
                                 (((1,), (1,)), ((), ())),
                                 preferred_element_type=jnp.float32)
        o_ref[0, s] = (r2.reshape(_OUT, _CBLK, _OUT).transpose(1, 0, 2)
                       .reshape(_CBLK, _OUT * _OUT))


def kernel(encoder_features, depths, intrinsics, masks):
    del depths, intrinsics
    n, c, _, _ = encoder_features.shape
    ms = (masks.reshape(n, _S, _HW, 4, 4 * _HW).transpose(0, 1, 3, 2, 4)
          .astype(jnp.int8))
    grid = (n, c // _CBLK)
    out = pl.pallas_call(
        _crop_kernel,
        grid=grid,
        in_specs=[
            pl.BlockSpec((1, _S, 1, _HW, 4 * _HW), lambda b, cb: (b, 0, 0, 0, 0)),
            pl.BlockSpec((1, _CBLK, _HW, _HW), lambda b, cb: (b, cb, 0, 0)),
        ],
        out_specs=pl.BlockSpec((1, _S, _CBLK, _OUT * _OUT),
                               lambda b, cb: (b, 0, cb, 0)),
        out_shape=jax.ShapeDtypeStruct((n, _S, c, _OUT * _OUT), jnp.float32),
    )(ms, encoder_features)
    return out.reshape(n, _S, c, _OUT, _OUT)
